# baseline (speedup 1.0000x reference)
import functools

import jax
import jax.numpy as jnp
from jax import lax
from jax.experimental import pallas as pl
from jax.experimental.pallas import tpu as pltpu
from jax.experimental.pallas import tpu_sc as plsc

NC = 2
NS = 16
_W = NC * NS
_D = 4
_F = 64


def _batch_size(epw):
  for b in range(128, 15, -1):
    if epw % b == 0 and b % 16 == 0:
      return b
  return 128


def _mesh():
  return plsc.VectorSubcoreMesh(
      core_axis_name="c", subcore_axis_name="s",
      num_cores=NC, num_subcores=NS)


def _row_split(n_nodes):
  ra = (n_nodes // NS) // 8 * 8
  return ra, n_nodes - NS * ra


def _sc_degree(dst2, dst_t, n_nodes):
  _, nb, b = dst2.shape
  tail = dst_t.shape[1] if dst_t is not None else 0
  ra, res = _row_split(n_nodes)

  scr = [pltpu.VMEM((nb, b), jnp.int32)]
  if tail:
    scr.append(pltpu.VMEM((tail,), jnp.int32))
  scr += [
      pltpu.VMEM((b, 16), jnp.float32),
      pltpu.VMEM_SHARED((n_nodes, 16), jnp.float32),
      pltpu.SemaphoreType.DMA,
  ]

  @functools.partial(
      pl.kernel,
      out_type=jax.ShapeDtypeStruct((NC, n_nodes, 16), jnp.float32),
      mesh=_mesh(),
      scratch_types=scr,
  )
  def deg_kernel(*refs):
    it = iter(refs)
    dst_hbm = next(it)
    dstt_hbm = next(it) if tail else None
    out_hbm = next(it)
    didx = next(it)
    didx_t = next(it) if tail else None
    ones_v = next(it)
    cnt_sh = next(it)
    sem = next(it)
    cid = lax.axis_index("c")
    sid = lax.axis_index("s")
    wid = sid * NC + cid
    tb = sid * ra

    def fill_row(i, carry):
      ones_v[i, :] = jnp.full((16,), 1.0, jnp.float32)
      return carry
    lax.fori_loop(0, b, fill_row, 0)
    pltpu.sync_copy(dst_hbm.at[wid], didx)
    if tail:
      pltpu.sync_copy(dstt_hbm.at[wid], didx_t)

    done = 0
    while done < ra:
      sz = min(b, ra - done)
      pltpu.sync_copy(ones_v.at[pl.ds(0, sz)],
                      cnt_sh.at[pl.ds(tb + done, sz)])
      done += sz
    if res:
      @pl.when(sid == NS - 1)
      def _():
        pltpu.sync_copy(ones_v.at[pl.ds(0, res)],
                        cnt_sh.at[pl.ds(NS * ra, res)])
    plsc.subcore_barrier()

    def batch(i, carry):
      pltpu.async_copy(ones_v, cnt_sh.at[didx.at[i]], sem, add=True)
      @pl.when(i >= _D)
      def _():
        pltpu.make_async_copy(ones_v, cnt_sh.at[didx.at[i]], sem).wait()
      return carry
    lax.fori_loop(0, nb, batch, 0)
    for d in range(min(_D, nb)):
      pltpu.make_async_copy(ones_v, cnt_sh.at[didx.at[d]], sem).wait()
    if tail:
      pltpu.sync_copy(ones_v.at[pl.ds(0, tail)], cnt_sh.at[didx_t], add=True)
    plsc.subcore_barrier()

    pltpu.sync_copy(cnt_sh.at[pl.ds(tb, ra)],
                    out_hbm.at[cid, pl.ds(tb, ra)])
    if res:
      @pl.when(sid == NS - 1)
      def _():
        pltpu.sync_copy(cnt_sh.at[pl.ds(NS * ra, res)],
                        out_hbm.at[cid, pl.ds(NS * ra, res)])

  args = (dst2, dst_t) if tail else (dst2,)
  return deg_kernel(*args)


def _sc_edge_agg(src2, dst2, src_t, dst_t, hp3):
  npass, n_nodes, f = hp3.shape
  _, nb, b = src2.shape
  tail = src_t.shape[1] if src_t is not None else 0
  ra, res = _row_split(n_nodes)
  kmain = nb // _D
  rem = nb % _D

  scr = [pltpu.VMEM((nb, b), jnp.int32), pltpu.VMEM((nb, b), jnp.int32)]
  if tail:
    scr += [pltpu.VMEM((tail,), jnp.int32), pltpu.VMEM((tail,), jnp.int32)]
  scr += [
      [pltpu.VMEM((b, f), jnp.float32)] * _D,
      [pltpu.SemaphoreType.DMA] * _D,
      [pltpu.SemaphoreType.DMA] * _D,
      pltpu.VMEM_SHARED((n_nodes, f), jnp.float32),
  ]

  @functools.partial(
      pl.kernel,
      out_type=jax.ShapeDtypeStruct((NC, npass, n_nodes, f), jnp.float32),
      mesh=_mesh(),
      compiler_params=pltpu.CompilerParams(use_tc_tiling_on_sc=False),
      scratch_types=scr,
  )
  def agg_kernel(*refs):
    it = iter(refs)
    src_hbm = next(it)
    dst_hbm = next(it)
    srct_hbm = next(it) if tail else None
    dstt_hbm = next(it) if tail else None
    hp_hbm = next(it)
    out_hbm = next(it)
    sidx = next(it)
    didx = next(it)
    sidx_t = next(it) if tail else None
    didx_t = next(it) if tail else None
    rows = next(it)
    gsem = next(it)
    ssem = next(it)
    agg_sh = next(it)
    cid = lax.axis_index("c")
    sid = lax.axis_index("s")
    wid = sid * NC + cid
    tb = sid * ra

    pltpu.sync_copy(src_hbm.at[wid], sidx)
    pltpu.sync_copy(dst_hbm.at[wid], didx)
    if tail:
      pltpu.sync_copy(srct_hbm.at[wid], sidx_t)
      pltpu.sync_copy(dstt_hbm.at[wid], didx_t)

    for p in range(npass):
      hview = hp_hbm.at[p]

      pltpu.sync_copy(hview.at[pl.ds(tb, ra)], agg_sh.at[pl.ds(tb, ra)])
      if res:
        @pl.when(sid == NS - 1)
        def _():
          pltpu.sync_copy(hview.at[pl.ds(NS * ra, res)],
                          agg_sh.at[pl.ds(NS * ra, res)])
      plsc.subcore_barrier()

      def start_gather(i, d):
        pltpu.async_copy(hview.at[sidx.at[i]], rows[d], gsem[d])

      def wait_gather(i, d):
        pltpu.make_async_copy(hview.at[sidx.at[i]], rows[d], gsem[d]).wait()

      def start_scatter(i, d):
        pltpu.async_copy(rows[d], agg_sh.at[didx.at[i]], ssem[d], add=True)

      def wait_scatter(i, d):
        pltpu.make_async_copy(rows[d], agg_sh.at[didx.at[i]], ssem[d]).wait()

      for d in range(min(_D, nb)):
        start_gather(d, d)

      def kbody(k, carry):
        for d in range(_D):
          i = k * _D + d
          wait_gather(i, d)
          start_scatter(i, d)
          @pl.when(i + _D < nb)
          def _():
            wait_scatter(i, d)
            start_gather(i + _D, d)
        return carry
      lax.fori_loop(0, kmain, kbody, 0)
      for d in range(rem):
        i = kmain * _D + d
        wait_gather(i, d)
        start_scatter(i, d)
      for d in range(min(_D, nb)):
        wait_scatter(0, d)
      if tail:
        pltpu.async_copy(hview.at[sidx_t], rows[0].at[pl.ds(0, tail)],
                         gsem[0]).wait()
        pltpu.sync_copy(rows[0].at[pl.ds(0, tail)], agg_sh.at[didx_t],
                        add=True)
      plsc.subcore_barrier()

      pltpu.sync_copy(agg_sh.at[pl.ds(tb, ra)],
                      out_hbm.at[cid, p, pl.ds(tb, ra)])
      if res:
        @pl.when(sid == NS - 1)
        def _():
          pltpu.sync_copy(agg_sh.at[pl.ds(NS * ra, res)],
                          out_hbm.at[cid, p, pl.ds(NS * ra, res)])
      if p + 1 < npass:
        plsc.subcore_barrier()

  args = ((src2, dst2, src_t, dst_t, hp3) if tail
          else (src2, dst2, hp3))
  return agg_kernel(*args)


_TC_PARAMS = pltpu.CompilerParams(
    dimension_semantics=("arbitrary",))


def _tc_first(x, w1, deg_parts, blk):
  n, nf = x.shape
  h = w1.shape[1]
  npass = h // _F

  def body(deg_ref, x_ref, w_ref, dinv_ref, h1p_ref):
    d = deg_ref[0, :, 0:1] + deg_ref[1, :, 0:1] - 1.0
    dinv = lax.rsqrt(d)
    dinv_ref[...] = dinv
    r = jnp.dot(x_ref[...], w_ref[...],
                preferred_element_type=jnp.float32) * dinv
    for p in range(npass):
      h1p_ref[p] = r[:, p * _F:(p + 1) * _F]

  grid = (n // blk,)
  return pl.pallas_call(
      body,
      grid=grid,
      in_specs=[
          pl.BlockSpec((NC, blk, 16), lambda i: (0, i, 0)),
          pl.BlockSpec((blk, nf), lambda i: (i, 0)),
          pl.BlockSpec((nf, h), lambda i: (0, 0)),
      ],
      out_specs=[
          pl.BlockSpec((blk, 1), lambda i: (i, 0)),
          pl.BlockSpec((npass, blk, _F), lambda i: (0, i, 0)),
      ],
      out_shape=[
          jax.ShapeDtypeStruct((n, 1), jnp.float32),
          jax.ShapeDtypeStruct((npass, n, _F), jnp.float32),
      ],
      compiler_params=_TC_PARAMS,
  )(deg_parts, x, w1)


def _tc_mid(agg1, h1p, dinv, b1, w2, blk):
  npass, n, _ = h1p.shape
  c = w2.shape[1]

  def body(a_ref, hp_ref, dinv_ref, b_ref, w_ref, h2p_ref):
    s = jnp.concatenate(
        [a_ref[0, p] + a_ref[1, p] - hp_ref[p] for p in range(npass)],
        axis=1)
    dinv = dinv_ref[...]
    h1 = jnp.maximum(s * dinv + b_ref[...], 0.0)
    h2p_ref[...] = jnp.dot(h1, w_ref[...],
                           preferred_element_type=jnp.float32) * dinv

  grid = (n // blk,)
  return pl.pallas_call(
      body,
      grid=grid,
      in_specs=[
          pl.BlockSpec((NC, npass, blk, _F), lambda i: (0, 0, i, 0)),
          pl.BlockSpec((npass, blk, _F), lambda i: (0, i, 0)),
          pl.BlockSpec((blk, 1), lambda i: (i, 0)),
          pl.BlockSpec((1, npass * _F), lambda i: (0, 0)),
          pl.BlockSpec((npass * _F, c), lambda i: (0, 0)),
      ],
      out_specs=pl.BlockSpec((blk, c), lambda i: (i, 0)),
      out_shape=jax.ShapeDtypeStruct((n, c), jnp.float32),
      compiler_params=_TC_PARAMS,
  )(agg1, h1p, dinv, b1, w2)


def _tc_last(agg2, h2p, dinv, b2, blk):
  n, c = h2p.shape

  def body(a_ref, hp_ref, dinv_ref, b_ref, fin_ref, logp_ref):
    s = a_ref[0] + a_ref[1] - hp_ref[...]
    fin = s * dinv_ref[...] + b_ref[...]
    m = jnp.max(fin, axis=1, keepdims=True)
    shifted = fin - m
    lse = jnp.log(jnp.sum(jnp.exp(shifted), axis=1, keepdims=True))
    fin_ref[...] = fin
    logp_ref[...] = shifted - lse

  grid = (n // blk,)
  return pl.pallas_call(
      body,
      grid=grid,
      in_specs=[
          pl.BlockSpec((NC, blk, c), lambda i: (0, i, 0)),
          pl.BlockSpec((blk, c), lambda i: (i, 0)),
          pl.BlockSpec((blk, 1), lambda i: (i, 0)),
          pl.BlockSpec((1, c), lambda i: (0, 0)),
      ],
      out_specs=[
          pl.BlockSpec((blk, c), lambda i: (i, 0)),
          pl.BlockSpec((blk, c), lambda i: (i, 0)),
      ],
      out_shape=[
          jax.ShapeDtypeStruct((n, c), jnp.float32),
          jax.ShapeDtypeStruct((n, c), jnp.float32),
      ],
      compiler_params=_TC_PARAMS,
  )(agg2, h2p, dinv, b2)


def kernel(x, edge_index, W1, b1, W2, b2):
  n = x.shape[0]
  e = edge_index.shape[1]
  src = edge_index[0]
  dst = edge_index[1]
  blk = 1000 if n % 1000 == 0 else 8

  epw = e // _W
  b = _batch_size(epw)
  nb = epw // b
  if nb * b == epw:
    src2 = src.reshape(_W, nb, b)
    dst2 = dst.reshape(_W, nb, b)
    src_t = dst_t = None
  else:
    src_w = src.reshape(_W, epw)
    dst_w = dst.reshape(_W, epw)
    src2 = src_w[:, :nb * b].reshape(_W, nb, b)
    dst2 = dst_w[:, :nb * b].reshape(_W, nb, b)
    src_t = src_w[:, nb * b:]
    dst_t = dst_w[:, nb * b:]

  deg_parts = _sc_degree(dst2, dst_t, n)
  dinv, h1p = _tc_first(x, W1, deg_parts, blk)
  agg1 = _sc_edge_agg(src2, dst2, src_t, dst_t, h1p)
  h2p = _tc_mid(agg1, h1p, dinv, b1.reshape(1, -1), W2, blk)
  c = h2p.shape[1]
  agg2 = _sc_edge_agg(src2, dst2, src_t, dst_t, h2p.reshape(-1, n, _F))
  final, logp = _tc_last(agg2.reshape(NC, n, c), h2p, dinv,
                         b2.reshape(1, -1), blk)
  return (final, logp)


# speedup vs baseline: 1.0139x; 1.0139x over previous
import functools

import jax
import jax.numpy as jnp
from jax import lax
from jax.experimental import pallas as pl
from jax.experimental.pallas import tpu as pltpu
from jax.experimental.pallas import tpu_sc as plsc

NC = 2
NS = 16
_W = NC * NS
_D = 6
_F = 64


def _batch_size(epw):
  if epw % 128 == 0:
    return 128
  for b in (128,):
    return b
  return 128


def _mesh():
  return plsc.VectorSubcoreMesh(
      core_axis_name="c", subcore_axis_name="s",
      num_cores=NC, num_subcores=NS)


def _row_split(n_nodes):
  ra = (n_nodes // NS) // 8 * 8
  return ra, n_nodes - NS * ra


def _sc_degree(dst2, dst_t, n_nodes):
  _, nb, b = dst2.shape
  tail = dst_t.shape[1] if dst_t is not None else 0
  ra, res = _row_split(n_nodes)

  scr = [pltpu.VMEM((nb, b), jnp.int32)]
  if tail:
    scr.append(pltpu.VMEM((tail,), jnp.int32))
  scr += [
      pltpu.VMEM((b, 16), jnp.float32),
      pltpu.VMEM_SHARED((n_nodes, 16), jnp.float32),
      pltpu.SemaphoreType.DMA,
  ]

  @functools.partial(
      pl.kernel,
      out_type=jax.ShapeDtypeStruct((NC, n_nodes, 16), jnp.float32),
      mesh=_mesh(),
      scratch_types=scr,
  )
  def deg_kernel(*refs):
    it = iter(refs)
    dst_hbm = next(it)
    dstt_hbm = next(it) if tail else None
    out_hbm = next(it)
    didx = next(it)
    didx_t = next(it) if tail else None
    ones_v = next(it)
    cnt_sh = next(it)
    sem = next(it)
    cid = lax.axis_index("c")
    sid = lax.axis_index("s")
    wid = sid * NC + cid
    tb = sid * ra

    def fill_row(i, carry):
      ones_v[i, :] = jnp.full((16,), 1.0, jnp.float32)
      return carry
    lax.fori_loop(0, b, fill_row, 0)
    pltpu.sync_copy(dst_hbm.at[wid], didx)
    if tail:
      pltpu.sync_copy(dstt_hbm.at[wid], didx_t)

    done = 0
    while done < ra:
      sz = min(b, ra - done)
      pltpu.sync_copy(ones_v.at[pl.ds(0, sz)],
                      cnt_sh.at[pl.ds(tb + done, sz)])
      done += sz
    if res:
      @pl.when(sid == NS - 1)
      def _():
        pltpu.sync_copy(ones_v.at[pl.ds(0, res)],
                        cnt_sh.at[pl.ds(NS * ra, res)])
    plsc.subcore_barrier()

    def batch(i, carry):
      pltpu.async_copy(ones_v, cnt_sh.at[didx.at[i]], sem, add=True)
      @pl.when(i >= _D)
      def _():
        pltpu.make_async_copy(ones_v, cnt_sh.at[didx.at[i]], sem).wait()
      return carry
    lax.fori_loop(0, nb, batch, 0)
    for d in range(min(_D, nb)):
      pltpu.make_async_copy(ones_v, cnt_sh.at[didx.at[d]], sem).wait()
    if tail:
      pltpu.sync_copy(ones_v.at[pl.ds(0, tail)], cnt_sh.at[didx_t], add=True)
    plsc.subcore_barrier()

    pltpu.sync_copy(cnt_sh.at[pl.ds(tb, ra)],
                    out_hbm.at[cid, pl.ds(tb, ra)])
    if res:
      @pl.when(sid == NS - 1)
      def _():
        pltpu.sync_copy(cnt_sh.at[pl.ds(NS * ra, res)],
                        out_hbm.at[cid, pl.ds(NS * ra, res)])

  args = (dst2, dst_t) if tail else (dst2,)
  return deg_kernel(*args)


def _sc_edge_agg(src2, dst2, src_t, dst_t, hp3):
  npass, n_nodes, f = hp3.shape
  _, nb, b = src2.shape
  tail = src_t.shape[1] if src_t is not None else 0
  ra, res = _row_split(n_nodes)
  kmain = nb // _D
  rem = nb % _D

  scr = [pltpu.VMEM((nb, b), jnp.int32), pltpu.VMEM((nb, b), jnp.int32)]
  if tail:
    scr += [pltpu.VMEM((tail,), jnp.int32), pltpu.VMEM((tail,), jnp.int32)]
  scr += [
      [pltpu.VMEM((b, f), jnp.float32)] * _D,
      [pltpu.SemaphoreType.DMA] * _D,
      [pltpu.SemaphoreType.DMA] * _D,
      pltpu.VMEM_SHARED((n_nodes, f), jnp.float32),
  ]

  @functools.partial(
      pl.kernel,
      out_type=jax.ShapeDtypeStruct((NC, npass, n_nodes, f), jnp.float32),
      mesh=_mesh(),
      compiler_params=pltpu.CompilerParams(use_tc_tiling_on_sc=False),
      scratch_types=scr,
  )
  def agg_kernel(*refs):
    it = iter(refs)
    src_hbm = next(it)
    dst_hbm = next(it)
    srct_hbm = next(it) if tail else None
    dstt_hbm = next(it) if tail else None
    hp_hbm = next(it)
    out_hbm = next(it)
    sidx = next(it)
    didx = next(it)
    sidx_t = next(it) if tail else None
    didx_t = next(it) if tail else None
    rows = next(it)
    gsem = next(it)
    ssem = next(it)
    agg_sh = next(it)
    cid = lax.axis_index("c")
    sid = lax.axis_index("s")
    wid = sid * NC + cid
    tb = sid * ra

    pltpu.sync_copy(src_hbm.at[wid], sidx)
    pltpu.sync_copy(dst_hbm.at[wid], didx)
    if tail:
      pltpu.sync_copy(srct_hbm.at[wid], sidx_t)
      pltpu.sync_copy(dstt_hbm.at[wid], didx_t)

    for p in range(npass):
      hview = hp_hbm.at[p]

      pltpu.sync_copy(hview.at[pl.ds(tb, ra)], agg_sh.at[pl.ds(tb, ra)])
      if res:
        @pl.when(sid == NS - 1)
        def _():
          pltpu.sync_copy(hview.at[pl.ds(NS * ra, res)],
                          agg_sh.at[pl.ds(NS * ra, res)])
      plsc.subcore_barrier()

      def start_gather(i, d):
        pltpu.async_copy(hview.at[sidx.at[i]], rows[d], gsem[d])

      def wait_gather(i, d):
        pltpu.make_async_copy(hview.at[sidx.at[i]], rows[d], gsem[d]).wait()

      def start_scatter(i, d):
        pltpu.async_copy(rows[d], agg_sh.at[didx.at[i]], ssem[d], add=True)

      def wait_scatter(i, d):
        pltpu.make_async_copy(rows[d], agg_sh.at[didx.at[i]], ssem[d]).wait()

      for d in range(min(_D, nb)):
        start_gather(d, d)

      def kbody(k, carry):
        for d in range(_D):
          i = k * _D + d
          wait_gather(i, d)
          start_scatter(i, d)
          @pl.when(i + _D < nb)
          def _():
            wait_scatter(i, d)
            start_gather(i + _D, d)
        return carry
      lax.fori_loop(0, kmain, kbody, 0)
      for d in range(rem):
        i = kmain * _D + d
        wait_gather(i, d)
        start_scatter(i, d)
      for d in range(min(_D, nb)):
        wait_scatter(0, d)
      if tail:
        pltpu.async_copy(hview.at[sidx_t], rows[0].at[pl.ds(0, tail)],
                         gsem[0]).wait()
        pltpu.sync_copy(rows[0].at[pl.ds(0, tail)], agg_sh.at[didx_t],
                        add=True)
      plsc.subcore_barrier()

      pltpu.sync_copy(agg_sh.at[pl.ds(tb, ra)],
                      out_hbm.at[cid, p, pl.ds(tb, ra)])
      if res:
        @pl.when(sid == NS - 1)
        def _():
          pltpu.sync_copy(agg_sh.at[pl.ds(NS * ra, res)],
                          out_hbm.at[cid, p, pl.ds(NS * ra, res)])
      if p + 1 < npass:
        plsc.subcore_barrier()

  args = ((src2, dst2, src_t, dst_t, hp3) if tail
          else (src2, dst2, hp3))
  return agg_kernel(*args)


_TC_PARAMS = pltpu.CompilerParams(
    dimension_semantics=("arbitrary",))


def _tc_first(x, w1, deg_parts, blk):
  n, nf = x.shape
  h = w1.shape[1]
  npass = h // _F

  def body(deg_ref, x_ref, w_ref, dinv_ref, h1p_ref):
    d = deg_ref[0, :, 0:1] + deg_ref[1, :, 0:1] - 1.0
    dinv = lax.rsqrt(d)
    dinv_ref[...] = dinv
    r = jnp.dot(x_ref[...], w_ref[...],
                preferred_element_type=jnp.float32) * dinv
    for p in range(npass):
      h1p_ref[p] = r[:, p * _F:(p + 1) * _F]

  grid = (n // blk,)
  return pl.pallas_call(
      body,
      grid=grid,
      in_specs=[
          pl.BlockSpec((NC, blk, 16), lambda i: (0, i, 0)),
          pl.BlockSpec((blk, nf), lambda i: (i, 0)),
          pl.BlockSpec((nf, h), lambda i: (0, 0)),
      ],
      out_specs=[
          pl.BlockSpec((blk, 1), lambda i: (i, 0)),
          pl.BlockSpec((npass, blk, _F), lambda i: (0, i, 0)),
      ],
      out_shape=[
          jax.ShapeDtypeStruct((n, 1), jnp.float32),
          jax.ShapeDtypeStruct((npass, n, _F), jnp.float32),
      ],
      compiler_params=_TC_PARAMS,
  )(deg_parts, x, w1)


def _tc_mid(agg1, h1p, dinv, b1, w2, blk):
  npass, n, _ = h1p.shape
  c = w2.shape[1]

  def body(a_ref, hp_ref, dinv_ref, b_ref, w_ref, h2p_ref):
    s = jnp.concatenate(
        [a_ref[0, p] + a_ref[1, p] - hp_ref[p] for p in range(npass)],
        axis=1)
    dinv = dinv_ref[...]
    h1 = jnp.maximum(s * dinv + b_ref[...], 0.0)
    h2p_ref[...] = jnp.dot(h1, w_ref[...],
                           preferred_element_type=jnp.float32) * dinv

  grid = (n // blk,)
  return pl.pallas_call(
      body,
      grid=grid,
      in_specs=[
          pl.BlockSpec((NC, npass, blk, _F), lambda i: (0, 0, i, 0)),
          pl.BlockSpec((npass, blk, _F), lambda i: (0, i, 0)),
          pl.BlockSpec((blk, 1), lambda i: (i, 0)),
          pl.BlockSpec((1, npass * _F), lambda i: (0, 0)),
          pl.BlockSpec((npass * _F, c), lambda i: (0, 0)),
      ],
      out_specs=pl.BlockSpec((blk, c), lambda i: (i, 0)),
      out_shape=jax.ShapeDtypeStruct((n, c), jnp.float32),
      compiler_params=_TC_PARAMS,
  )(agg1, h1p, dinv, b1, w2)


def _tc_last(agg2, h2p, dinv, b2, blk):
  n, c = h2p.shape

  def body(a_ref, hp_ref, dinv_ref, b_ref, fin_ref, logp_ref):
    s = a_ref[0] + a_ref[1] - hp_ref[...]
    fin = s * dinv_ref[...] + b_ref[...]
    m = jnp.max(fin, axis=1, keepdims=True)
    shifted = fin - m
    lse = jnp.log(jnp.sum(jnp.exp(shifted), axis=1, keepdims=True))
    fin_ref[...] = fin
    logp_ref[...] = shifted - lse

  grid = (n // blk,)
  return pl.pallas_call(
      body,
      grid=grid,
      in_specs=[
          pl.BlockSpec((NC, blk, c), lambda i: (0, i, 0)),
          pl.BlockSpec((blk, c), lambda i: (i, 0)),
          pl.BlockSpec((blk, 1), lambda i: (i, 0)),
          pl.BlockSpec((1, c), lambda i: (0, 0)),
      ],
      out_specs=[
          pl.BlockSpec((blk, c), lambda i: (i, 0)),
          pl.BlockSpec((blk, c), lambda i: (i, 0)),
      ],
      out_shape=[
          jax.ShapeDtypeStruct((n, c), jnp.float32),
          jax.ShapeDtypeStruct((n, c), jnp.float32),
      ],
      compiler_params=_TC_PARAMS,
  )(agg2, h2p, dinv, b2)


def kernel(x, edge_index, W1, b1, W2, b2):
  n = x.shape[0]
  e = edge_index.shape[1]
  src = edge_index[0]
  dst = edge_index[1]
  blk = 1000 if n % 1000 == 0 else 8

  epw = e // _W
  b = _batch_size(epw)
  nb = epw // b
  if nb * b == epw:
    src2 = src.reshape(_W, nb, b)
    dst2 = dst.reshape(_W, nb, b)
    src_t = dst_t = None
  else:
    src_w = src.reshape(_W, epw)
    dst_w = dst.reshape(_W, epw)
    src2 = src_w[:, :nb * b].reshape(_W, nb, b)
    dst2 = dst_w[:, :nb * b].reshape(_W, nb, b)
    src_t = src_w[:, nb * b:]
    dst_t = dst_w[:, nb * b:]

  deg_parts = _sc_degree(dst2, dst_t, n)
  dinv, h1p = _tc_first(x, W1, deg_parts, blk)
  agg1 = _sc_edge_agg(src2, dst2, src_t, dst_t, h1p)
  h2p = _tc_mid(agg1, h1p, dinv, b1.reshape(1, -1), W2, blk)
  c = h2p.shape[1]
  agg2 = _sc_edge_agg(src2, dst2, src_t, dst_t, h2p.reshape(-1, n, _F))
  final, logp = _tc_last(agg2.reshape(NC, n, c), h2p, dinv,
                         b2.reshape(1, -1), blk)
  return (final, logp)


# speedup vs baseline: 1.1869x; 1.1706x over previous
import functools

import jax
import jax.numpy as jnp
from jax import lax
from jax.experimental import pallas as pl
from jax.experimental.pallas import tpu as pltpu
from jax.experimental.pallas import tpu_sc as plsc

NC = 2
NS = 16
_W = NC * NS
_D = 6
_F = 64


def _batch_size(epw):
  if epw % 128 == 0:
    return 128
  for b in (128,):
    return b
  return 128


def _mesh():
  return plsc.VectorSubcoreMesh(
      core_axis_name="c", subcore_axis_name="s",
      num_cores=NC, num_subcores=NS)


def _row_split(n_nodes):
  ra = (n_nodes // NS) // 8 * 8
  return ra, n_nodes - NS * ra


def _sc_degree(dst2, dst_t, n_nodes):
  _, nb, b = dst2.shape
  tail = dst_t.shape[1] if dst_t is not None else 0
  ra, res = _row_split(n_nodes)

  scr = [pltpu.VMEM((nb, b), jnp.int32)]
  if tail:
    scr.append(pltpu.VMEM((tail,), jnp.int32))
  scr += [
      pltpu.VMEM((b, 16), jnp.float32),
      pltpu.VMEM_SHARED((n_nodes, 16), jnp.float32),
      pltpu.SemaphoreType.DMA,
  ]

  @functools.partial(
      pl.kernel,
      out_type=jax.ShapeDtypeStruct((NC, n_nodes, 16), jnp.float32),
      mesh=_mesh(),
      scratch_types=scr,
  )
  def deg_kernel(*refs):
    it = iter(refs)
    dst_hbm = next(it)
    dstt_hbm = next(it) if tail else None
    out_hbm = next(it)
    didx = next(it)
    didx_t = next(it) if tail else None
    ones_v = next(it)
    cnt_sh = next(it)
    sem = next(it)
    cid = lax.axis_index("c")
    sid = lax.axis_index("s")
    wid = sid * NC + cid
    tb = sid * ra

    def fill_row(i, carry):
      ones_v[i, :] = jnp.full((16,), 1.0, jnp.float32)
      return carry
    lax.fori_loop(0, b, fill_row, 0)
    pltpu.sync_copy(dst_hbm.at[wid], didx)
    if tail:
      pltpu.sync_copy(dstt_hbm.at[wid], didx_t)

    done = 0
    while done < ra:
      sz = min(b, ra - done)
      pltpu.sync_copy(ones_v.at[pl.ds(0, sz)],
                      cnt_sh.at[pl.ds(tb + done, sz)])
      done += sz
    if res:
      @pl.when(sid == NS - 1)
      def _():
        pltpu.sync_copy(ones_v.at[pl.ds(0, res)],
                        cnt_sh.at[pl.ds(NS * ra, res)])
    plsc.subcore_barrier()

    def batch(i, carry):
      pltpu.async_copy(ones_v, cnt_sh.at[didx.at[i]], sem, add=True)
      @pl.when(i >= _D)
      def _():
        pltpu.make_async_copy(ones_v, cnt_sh.at[didx.at[i]], sem).wait()
      return carry
    lax.fori_loop(0, nb, batch, 0)
    for d in range(min(_D, nb)):
      pltpu.make_async_copy(ones_v, cnt_sh.at[didx.at[d]], sem).wait()
    if tail:
      pltpu.sync_copy(ones_v.at[pl.ds(0, tail)], cnt_sh.at[didx_t], add=True)
    plsc.subcore_barrier()

    pltpu.sync_copy(cnt_sh.at[pl.ds(tb, ra)],
                    out_hbm.at[cid, pl.ds(tb, ra)])
    if res:
      @pl.when(sid == NS - 1)
      def _():
        pltpu.sync_copy(cnt_sh.at[pl.ds(NS * ra, res)],
                        out_hbm.at[cid, pl.ds(NS * ra, res)])

  args = (dst2, dst_t) if tail else (dst2,)
  return deg_kernel(*args)


def _sc_edge_agg(src2, dst2, src_t, dst_t, hp3):
  npass, n_nodes, f = hp3.shape
  _, nb, b = src2.shape
  tail = src_t.shape[1] if src_t is not None else 0
  ra, res = _row_split(n_nodes)
  kmain = nb // _D
  rem = nb % _D

  scr = [pltpu.VMEM((nb, b), jnp.int32), pltpu.VMEM((nb, b), jnp.int32)]
  if tail:
    scr += [pltpu.VMEM((tail,), jnp.int32), pltpu.VMEM((tail,), jnp.int32)]
  scr += [
      [pltpu.VMEM((b, f), jnp.float32)] * _D,
      [pltpu.SemaphoreType.DMA] * _D,
      [pltpu.SemaphoreType.DMA] * _D,
      pltpu.VMEM_SHARED((n_nodes, f), jnp.float32),
  ]

  @functools.partial(
      pl.kernel,
      out_type=jax.ShapeDtypeStruct((NC, npass, n_nodes, f), jnp.float32),
      mesh=_mesh(),
      compiler_params=pltpu.CompilerParams(use_tc_tiling_on_sc=False),
      scratch_types=scr,
  )
  def agg_kernel(*refs):
    it = iter(refs)
    src_hbm = next(it)
    dst_hbm = next(it)
    srct_hbm = next(it) if tail else None
    dstt_hbm = next(it) if tail else None
    hp_hbm = next(it)
    out_hbm = next(it)
    sidx = next(it)
    didx = next(it)
    sidx_t = next(it) if tail else None
    didx_t = next(it) if tail else None
    rows = next(it)
    gsem = next(it)
    ssem = next(it)
    agg_sh = next(it)
    cid = lax.axis_index("c")
    sid = lax.axis_index("s")
    wid = sid * NC + cid
    tb = sid * ra

    pltpu.sync_copy(src_hbm.at[wid], sidx)
    pltpu.sync_copy(dst_hbm.at[wid], didx)
    if tail:
      pltpu.sync_copy(srct_hbm.at[wid], sidx_t)
      pltpu.sync_copy(dstt_hbm.at[wid], didx_t)

    for p in range(npass):
      hview = hp_hbm.at[p]

      pltpu.sync_copy(hview.at[pl.ds(tb, ra)], agg_sh.at[pl.ds(tb, ra)])
      if res:
        @pl.when(sid == NS - 1)
        def _():
          pltpu.sync_copy(hview.at[pl.ds(NS * ra, res)],
                          agg_sh.at[pl.ds(NS * ra, res)])
      plsc.subcore_barrier()

      def start_gather(i, d):
        pltpu.async_copy(hview.at[sidx.at[i]], rows[d], gsem[d])

      def wait_gather(i, d):
        pltpu.make_async_copy(hview.at[sidx.at[i]], rows[d], gsem[d]).wait()

      def start_scatter(i, d):
        pltpu.async_copy(rows[d], agg_sh.at[didx.at[i]], ssem[d], add=True)

      def wait_scatter(i, d):
        pltpu.make_async_copy(rows[d], agg_sh.at[didx.at[i]], ssem[d]).wait()

      for d in range(min(_D, nb)):
        start_gather(d, d)

      def kbody(k, carry):
        for d in range(_D):
          i = k * _D + d
          wait_gather(i, d)
          start_scatter(i, d)
          @pl.when(i + _D < nb)
          def _():
            wait_scatter(i, d)
            start_gather(i + _D, d)
        return carry
      lax.fori_loop(0, kmain, kbody, 0)
      for d in range(rem):
        i = kmain * _D + d
        wait_gather(i, d)
        start_scatter(i, d)
      for d in range(min(_D, nb)):
        wait_scatter(0, d)
      if tail:
        pltpu.async_copy(hview.at[sidx_t], rows[0].at[pl.ds(0, tail)],
                         gsem[0]).wait()
        pltpu.sync_copy(rows[0].at[pl.ds(0, tail)], agg_sh.at[didx_t],
                        add=True)
      plsc.subcore_barrier()

      pltpu.sync_copy(agg_sh.at[pl.ds(tb, ra)],
                      out_hbm.at[cid, p, pl.ds(tb, ra)])
      if res:
        @pl.when(sid == NS - 1)
        def _():
          pltpu.sync_copy(agg_sh.at[pl.ds(NS * ra, res)],
                          out_hbm.at[cid, p, pl.ds(NS * ra, res)])
      if p + 1 < npass:
        plsc.subcore_barrier()

  args = ((src2, dst2, src_t, dst_t, hp3) if tail
          else (src2, dst2, hp3))
  return agg_kernel(*args)


_TC_PARAMS = pltpu.CompilerParams(
    dimension_semantics=("arbitrary",))


def _tc_first(x_pk, w1bd, deg_pk, blkh):
  nh, nf2 = x_pk.shape
  npass = w1bd.shape[0]

  def body(deg_ref, x_ref, w_ref, dinv_ref, h1p_ref):
    de = deg_ref[0, :, 0:1] + deg_ref[1, :, 0:1] - 1.0
    do = deg_ref[0, :, 16:17] + deg_ref[1, :, 16:17] - 1.0
    dinv = jnp.concatenate(
        [jnp.broadcast_to(lax.rsqrt(de), (blkh, _F)),
         jnp.broadcast_to(lax.rsqrt(do), (blkh, _F))], axis=1)
    dinv_ref[...] = dinv
    for p in range(npass):
      h1p_ref[p] = jnp.dot(x_ref[...], w_ref[p],
                           preferred_element_type=jnp.float32) * dinv

  grid = (nh // blkh,)
  return pl.pallas_call(
      body,
      grid=grid,
      in_specs=[
          pl.BlockSpec((NC, blkh, 32), lambda i: (0, i, 0)),
          pl.BlockSpec((blkh, nf2), lambda i: (i, 0)),
          pl.BlockSpec((npass, nf2, 2 * _F), lambda i: (0, 0, 0)),
      ],
      out_specs=[
          pl.BlockSpec((blkh, 2 * _F), lambda i: (i, 0)),
          pl.BlockSpec((npass, blkh, 2 * _F), lambda i: (0, i, 0)),
      ],
      out_shape=[
          jax.ShapeDtypeStruct((nh, 2 * _F), jnp.float32),
          jax.ShapeDtypeStruct((npass, nh, 2 * _F), jnp.float32),
      ],
      compiler_params=_TC_PARAMS,
  )(deg_pk, x_pk, w1bd)


def _tc_mid(agg1_pk, h1p_pk, dinv_pk, b1pk, w2bd, blkh):
  npass, nh, _ = h1p_pk.shape

  def body(a_ref, hp_ref, dinv_ref, b_ref, w_ref, out_ref):
    dinv = dinv_ref[...]
    acc = None
    for p in range(npass):
      s = a_ref[0, p] + a_ref[1, p] - hp_ref[p]
      h1 = jnp.maximum(s * dinv + b_ref[p], 0.0)
      t = jnp.dot(h1, w_ref[p], preferred_element_type=jnp.float32)
      acc = t if acc is None else acc + t
    out_ref[...] = acc * dinv

  grid = (nh // blkh,)
  return pl.pallas_call(
      body,
      grid=grid,
      in_specs=[
          pl.BlockSpec((NC, npass, blkh, 2 * _F), lambda i: (0, 0, i, 0)),
          pl.BlockSpec((npass, blkh, 2 * _F), lambda i: (0, i, 0)),
          pl.BlockSpec((blkh, 2 * _F), lambda i: (i, 0)),
          pl.BlockSpec((npass, 1, 2 * _F), lambda i: (0, 0, 0)),
          pl.BlockSpec((npass, 2 * _F, 2 * _F), lambda i: (0, 0, 0)),
      ],
      out_specs=pl.BlockSpec((blkh, 2 * _F), lambda i: (i, 0)),
      out_shape=jax.ShapeDtypeStruct((nh, 2 * _F), jnp.float32),
      compiler_params=_TC_PARAMS,
  )(agg1_pk, h1p_pk, dinv_pk, b1pk, w2bd)


def _tc_last(agg2_pk, h2p_pk, dinv_pk, b2pk, blkh):
  nh, _ = h2p_pk.shape

  def lsm(f):
    m = jnp.max(f, axis=1, keepdims=True)
    sh = f - m
    return sh - jnp.log(jnp.sum(jnp.exp(sh), axis=1, keepdims=True))

  def body(a_ref, hp_ref, dinv_ref, b_ref, fin_ref, logp_ref):
    s = a_ref[0] + a_ref[1] - hp_ref[...]
    fin = s * dinv_ref[...] + b_ref[...]
    fin_ref[...] = fin
    logp_ref[...] = jnp.concatenate(
        [lsm(fin[:, :_F]), lsm(fin[:, _F:])], axis=1)

  grid = (nh // blkh,)
  return pl.pallas_call(
      body,
      grid=grid,
      in_specs=[
          pl.BlockSpec((NC, blkh, 2 * _F), lambda i: (0, i, 0)),
          pl.BlockSpec((blkh, 2 * _F), lambda i: (i, 0)),
          pl.BlockSpec((blkh, 2 * _F), lambda i: (i, 0)),
          pl.BlockSpec((1, 2 * _F), lambda i: (0, 0)),
      ],
      out_specs=[
          pl.BlockSpec((blkh, 2 * _F), lambda i: (i, 0)),
          pl.BlockSpec((blkh, 2 * _F), lambda i: (i, 0)),
      ],
      out_shape=[
          jax.ShapeDtypeStruct((nh, 2 * _F), jnp.float32),
          jax.ShapeDtypeStruct((nh, 2 * _F), jnp.float32),
      ],
      compiler_params=_TC_PARAMS,
  )(agg2_pk, h2p_pk, dinv_pk, b2pk)


def _block_diag2(w):
  k = w.shape[0]
  z = jnp.zeros((k, _F), jnp.float32)
  return jnp.concatenate(
      [jnp.concatenate([w, z], axis=1),
       jnp.concatenate([z, w], axis=1)], axis=0)


def kernel(x, edge_index, W1, b1, W2, b2):
  n = x.shape[0]
  e = edge_index.shape[1]
  nf = x.shape[1]
  src = edge_index[0]
  dst = edge_index[1]
  nh = n // 2
  blkh = 1000 if nh % 1000 == 0 else 8
  npass = W1.shape[1] // _F

  epw = e // _W
  b = _batch_size(epw)
  nb = epw // b
  if nb * b == epw:
    src2 = src.reshape(_W, nb, b)
    dst2 = dst.reshape(_W, nb, b)
    src_t = dst_t = None
  else:
    src_w = src.reshape(_W, epw)
    dst_w = dst.reshape(_W, epw)
    src2 = src_w[:, :nb * b].reshape(_W, nb, b)
    dst2 = dst_w[:, :nb * b].reshape(_W, nb, b)
    src_t = src_w[:, nb * b:]
    dst_t = dst_w[:, nb * b:]

  x_pk = x.reshape(nh, 2 * nf)
  w1bd = jnp.stack([_block_diag2(W1[:, p * _F:(p + 1) * _F])
                    for p in range(npass)])
  w2bd = jnp.stack([_block_diag2(W2[p * _F:(p + 1) * _F, :])
                    for p in range(npass)])
  b1pk = jnp.stack([jnp.concatenate([b1[p * _F:(p + 1) * _F]] * 2)
                    for p in range(npass)])[:, None, :]
  b2pk = jnp.concatenate([b2, b2]).reshape(1, 2 * _F)

  deg_parts = _sc_degree(dst2, dst_t, n)
  deg_pk = deg_parts.reshape(NC, nh, 32)
  dinv_pk, h1p_pk = _tc_first(x_pk, w1bd, deg_pk, blkh)
  agg1 = _sc_edge_agg(src2, dst2, src_t, dst_t,
                      h1p_pk.reshape(npass, n, _F))
  h2p_pk = _tc_mid(agg1.reshape(NC, npass, nh, 2 * _F), h1p_pk,
                   dinv_pk, b1pk, w2bd, blkh)
  agg2 = _sc_edge_agg(src2, dst2, src_t, dst_t, h2p_pk.reshape(1, n, _F))
  fin_pk, logp_pk = _tc_last(agg2.reshape(NC, nh, 2 * _F), h2p_pk,
                             dinv_pk, b2pk, blkh)
  return (fin_pk.reshape(n, _F), logp_pk.reshape(n, _F))


# speedup vs baseline: 1.2436x; 1.0478x over previous
import functools

import jax
import jax.numpy as jnp
from jax import lax
from jax.experimental import pallas as pl
from jax.experimental.pallas import tpu as pltpu
from jax.experimental.pallas import tpu_sc as plsc

NC = 2
NS = 16
_W = NC * NS
_D = 6
_F = 64


def _batch_size(epw):
  if epw % 128 == 0:
    return 128
  for b in (128,):
    return b
  return 128


def _mesh():
  return plsc.VectorSubcoreMesh(
      core_axis_name="c", subcore_axis_name="s",
      num_cores=NC, num_subcores=NS)


def _row_split(n_nodes):
  ra = (n_nodes // NS) // 8 * 8
  return ra, n_nodes - NS * ra


def _sc_degree(dst2, dst_t, n_nodes):
  _, nb, b = dst2.shape
  tail = dst_t.shape[1] if dst_t is not None else 0
  ra, res = _row_split(n_nodes)

  scr = [pltpu.VMEM((nb, b), jnp.int32)]
  if tail:
    scr.append(pltpu.VMEM((tail,), jnp.int32))
  scr += [
      pltpu.VMEM((b, 16), jnp.float32),
      pltpu.VMEM_SHARED((n_nodes, 16), jnp.float32),
      pltpu.SemaphoreType.DMA,
  ]

  @functools.partial(
      pl.kernel,
      out_type=jax.ShapeDtypeStruct((NC, n_nodes, 16), jnp.float32),
      mesh=_mesh(),
      compiler_params=pltpu.CompilerParams(use_tc_tiling_on_sc=False),
      scratch_types=scr,
  )
  def deg_kernel(*refs):
    it = iter(refs)
    dst_hbm = next(it)
    dstt_hbm = next(it) if tail else None
    out_hbm = next(it)
    didx = next(it)
    didx_t = next(it) if tail else None
    ones_v = next(it)
    cnt_sh = next(it)
    sem = next(it)
    cid = lax.axis_index("c")
    sid = lax.axis_index("s")
    wid = sid * NC + cid
    tb = sid * ra

    def fill_row(i, carry):
      ones_v[i, :] = jnp.full((16,), 1.0, jnp.float32)
      return carry
    lax.fori_loop(0, b, fill_row, 0)
    pltpu.sync_copy(dst_hbm.at[wid], didx)
    if tail:
      pltpu.sync_copy(dstt_hbm.at[wid], didx_t)

    done = 0
    while done < ra:
      sz = min(b, ra - done)
      pltpu.sync_copy(ones_v.at[pl.ds(0, sz)],
                      cnt_sh.at[pl.ds(tb + done, sz)])
      done += sz
    if res:
      @pl.when(sid == NS - 1)
      def _():
        pltpu.sync_copy(ones_v.at[pl.ds(0, res)],
                        cnt_sh.at[pl.ds(NS * ra, res)])
    plsc.subcore_barrier()

    def batch(i, carry):
      pltpu.async_copy(ones_v, cnt_sh.at[didx.at[i]], sem, add=True)
      @pl.when(i >= _D)
      def _():
        pltpu.make_async_copy(ones_v, cnt_sh.at[didx.at[i]], sem).wait()
      return carry
    lax.fori_loop(0, nb, batch, 0)
    for d in range(min(_D, nb)):
      pltpu.make_async_copy(ones_v, cnt_sh.at[didx.at[d]], sem).wait()
    if tail:
      pltpu.sync_copy(ones_v.at[pl.ds(0, tail)], cnt_sh.at[didx_t], add=True)
    plsc.subcore_barrier()

    pltpu.sync_copy(cnt_sh.at[pl.ds(tb, ra)],
                    out_hbm.at[cid, pl.ds(tb, ra)])
    if res:
      @pl.when(sid == NS - 1)
      def _():
        pltpu.sync_copy(cnt_sh.at[pl.ds(NS * ra, res)],
                        out_hbm.at[cid, pl.ds(NS * ra, res)])

  args = (dst2, dst_t) if tail else (dst2,)
  return deg_kernel(*args)


def _sc_edge_agg(src2, dst2, src_t, dst_t, hp3):
  npass, n_nodes, f = hp3.shape
  _, nb, b = src2.shape
  tail = src_t.shape[1] if src_t is not None else 0
  ra, res = _row_split(n_nodes)
  kmain = nb // _D
  rem = nb % _D

  scr = [pltpu.VMEM((nb, b), jnp.int32), pltpu.VMEM((nb, b), jnp.int32)]
  if tail:
    scr += [pltpu.VMEM((tail,), jnp.int32), pltpu.VMEM((tail,), jnp.int32)]
  scr += [
      [pltpu.VMEM((b, f), jnp.float32)] * _D,
      [pltpu.SemaphoreType.DMA] * _D,
      [pltpu.SemaphoreType.DMA] * _D,
      pltpu.VMEM_SHARED((n_nodes, f), jnp.float32),
  ]

  @functools.partial(
      pl.kernel,
      out_type=jax.ShapeDtypeStruct((NC, npass, n_nodes, f), jnp.float32),
      mesh=_mesh(),
      compiler_params=pltpu.CompilerParams(use_tc_tiling_on_sc=False),
      scratch_types=scr,
  )
  def agg_kernel(*refs):
    it = iter(refs)
    src_hbm = next(it)
    dst_hbm = next(it)
    srct_hbm = next(it) if tail else None
    dstt_hbm = next(it) if tail else None
    hp_hbm = next(it)
    out_hbm = next(it)
    sidx = next(it)
    didx = next(it)
    sidx_t = next(it) if tail else None
    didx_t = next(it) if tail else None
    rows = next(it)
    gsem = next(it)
    ssem = next(it)
    agg_sh = next(it)
    cid = lax.axis_index("c")
    sid = lax.axis_index("s")
    wid = sid * NC + cid
    tb = sid * ra

    pltpu.sync_copy(src_hbm.at[wid], sidx)
    pltpu.sync_copy(dst_hbm.at[wid], didx)
    if tail:
      pltpu.sync_copy(srct_hbm.at[wid], sidx_t)
      pltpu.sync_copy(dstt_hbm.at[wid], didx_t)

    for p in range(npass):
      hview = hp_hbm.at[p]

      pltpu.sync_copy(hview.at[pl.ds(tb, ra)], agg_sh.at[pl.ds(tb, ra)])
      if res:
        @pl.when(sid == NS - 1)
        def _():
          pltpu.sync_copy(hview.at[pl.ds(NS * ra, res)],
                          agg_sh.at[pl.ds(NS * ra, res)])
      plsc.subcore_barrier()

      def start_gather(i, d):
        pltpu.async_copy(hview.at[sidx.at[i]], rows[d], gsem[d])

      def wait_gather(i, d):
        pltpu.make_async_copy(hview.at[sidx.at[i]], rows[d], gsem[d]).wait()

      def start_scatter(i, d):
        pltpu.async_copy(rows[d], agg_sh.at[didx.at[i]], ssem[d], add=True)

      def wait_scatter(i, d):
        pltpu.make_async_copy(rows[d], agg_sh.at[didx.at[i]], ssem[d]).wait()

      for d in range(min(_D, nb)):
        start_gather(d, d)

      def kbody(k, carry):
        for d in range(_D):
          i = k * _D + d
          wait_gather(i, d)
          start_scatter(i, d)
          @pl.when(i + _D < nb)
          def _():
            wait_scatter(i, d)
            start_gather(i + _D, d)
        return carry
      lax.fori_loop(0, kmain, kbody, 0)
      for d in range(rem):
        i = kmain * _D + d
        wait_gather(i, d)
        start_scatter(i, d)
      for d in range(min(_D, nb)):
        wait_scatter(0, d)
      if tail:
        pltpu.async_copy(hview.at[sidx_t], rows[0].at[pl.ds(0, tail)],
                         gsem[0]).wait()
        pltpu.sync_copy(rows[0].at[pl.ds(0, tail)], agg_sh.at[didx_t],
                        add=True)
      plsc.subcore_barrier()

      pltpu.sync_copy(agg_sh.at[pl.ds(tb, ra)],
                      out_hbm.at[cid, p, pl.ds(tb, ra)])
      if res:
        @pl.when(sid == NS - 1)
        def _():
          pltpu.sync_copy(agg_sh.at[pl.ds(NS * ra, res)],
                          out_hbm.at[cid, p, pl.ds(NS * ra, res)])
      if p + 1 < npass:
        plsc.subcore_barrier()

  args = ((src2, dst2, src_t, dst_t, hp3) if tail
          else (src2, dst2, hp3))
  return agg_kernel(*args)


_TC_PARAMS = pltpu.CompilerParams(
    dimension_semantics=("arbitrary",))


def _tc_first(x_pk, w1bd, deg_pk, blkh):
  nh, nf2 = x_pk.shape
  npass = w1bd.shape[0]

  def body(deg_ref, x_ref, w_ref, dinv_ref, h1p_ref):
    de = deg_ref[0, :, 0:1] + deg_ref[1, :, 0:1] - 1.0
    do = deg_ref[0, :, 16:17] + deg_ref[1, :, 16:17] - 1.0
    dinv = jnp.concatenate(
        [jnp.broadcast_to(lax.rsqrt(de), (blkh, _F)),
         jnp.broadcast_to(lax.rsqrt(do), (blkh, _F))], axis=1)
    dinv_ref[...] = dinv
    for p in range(npass):
      h1p_ref[p] = jnp.dot(x_ref[...], w_ref[p],
                           preferred_element_type=jnp.float32) * dinv

  grid = (nh // blkh,)
  return pl.pallas_call(
      body,
      grid=grid,
      in_specs=[
          pl.BlockSpec((NC, blkh, 32), lambda i: (0, i, 0)),
          pl.BlockSpec((blkh, nf2), lambda i: (i, 0)),
          pl.BlockSpec((npass, nf2, 2 * _F), lambda i: (0, 0, 0)),
      ],
      out_specs=[
          pl.BlockSpec((blkh, 2 * _F), lambda i: (i, 0)),
          pl.BlockSpec((npass, blkh, 2 * _F), lambda i: (0, i, 0)),
      ],
      out_shape=[
          jax.ShapeDtypeStruct((nh, 2 * _F), jnp.float32),
          jax.ShapeDtypeStruct((npass, nh, 2 * _F), jnp.float32),
      ],
      compiler_params=_TC_PARAMS,
  )(deg_pk, x_pk, w1bd)


def _tc_mid(agg1_pk, h1p_pk, dinv_pk, b1pk, w2bd, blkh):
  npass, nh, _ = h1p_pk.shape

  def body(a_ref, hp_ref, dinv_ref, b_ref, w_ref, out_ref):
    dinv = dinv_ref[...]
    acc = None
    for p in range(npass):
      s = a_ref[0, p] + a_ref[1, p] - hp_ref[p]
      h1 = jnp.maximum(s * dinv + b_ref[p], 0.0)
      t = jnp.dot(h1, w_ref[p], preferred_element_type=jnp.float32)
      acc = t if acc is None else acc + t
    out_ref[...] = acc * dinv

  grid = (nh // blkh,)
  return pl.pallas_call(
      body,
      grid=grid,
      in_specs=[
          pl.BlockSpec((NC, npass, blkh, 2 * _F), lambda i: (0, 0, i, 0)),
          pl.BlockSpec((npass, blkh, 2 * _F), lambda i: (0, i, 0)),
          pl.BlockSpec((blkh, 2 * _F), lambda i: (i, 0)),
          pl.BlockSpec((npass, 1, 2 * _F), lambda i: (0, 0, 0)),
          pl.BlockSpec((npass, 2 * _F, 2 * _F), lambda i: (0, 0, 0)),
      ],
      out_specs=pl.BlockSpec((blkh, 2 * _F), lambda i: (i, 0)),
      out_shape=jax.ShapeDtypeStruct((nh, 2 * _F), jnp.float32),
      compiler_params=_TC_PARAMS,
  )(agg1_pk, h1p_pk, dinv_pk, b1pk, w2bd)


def _tc_last(agg2_pk, h2p_pk, dinv_pk, b2pk, blkh):
  nh, _ = h2p_pk.shape

  def lsm(f):
    m = jnp.max(f, axis=1, keepdims=True)
    sh = f - m
    return sh - jnp.log(jnp.sum(jnp.exp(sh), axis=1, keepdims=True))

  def body(a_ref, hp_ref, dinv_ref, b_ref, fin_ref, logp_ref):
    s = a_ref[0] + a_ref[1] - hp_ref[...]
    fin = s * dinv_ref[...] + b_ref[...]
    fin_ref[...] = fin
    logp_ref[...] = jnp.concatenate(
        [lsm(fin[:, :_F]), lsm(fin[:, _F:])], axis=1)

  grid = (nh // blkh,)
  return pl.pallas_call(
      body,
      grid=grid,
      in_specs=[
          pl.BlockSpec((NC, blkh, 2 * _F), lambda i: (0, i, 0)),
          pl.BlockSpec((blkh, 2 * _F), lambda i: (i, 0)),
          pl.BlockSpec((blkh, 2 * _F), lambda i: (i, 0)),
          pl.BlockSpec((1, 2 * _F), lambda i: (0, 0)),
      ],
      out_specs=[
          pl.BlockSpec((blkh, 2 * _F), lambda i: (i, 0)),
          pl.BlockSpec((blkh, 2 * _F), lambda i: (i, 0)),
      ],
      out_shape=[
          jax.ShapeDtypeStruct((nh, 2 * _F), jnp.float32),
          jax.ShapeDtypeStruct((nh, 2 * _F), jnp.float32),
      ],
      compiler_params=_TC_PARAMS,
  )(agg2_pk, h2p_pk, dinv_pk, b2pk)


def _block_diag2(w):
  k = w.shape[0]
  z = jnp.zeros((k, _F), jnp.float32)
  return jnp.concatenate(
      [jnp.concatenate([w, z], axis=1),
       jnp.concatenate([z, w], axis=1)], axis=0)


def kernel(x, edge_index, W1, b1, W2, b2):
  n = x.shape[0]
  e = edge_index.shape[1]
  nf = x.shape[1]
  src = edge_index[0]
  dst = edge_index[1]
  nh = n // 2
  blkh = 1000 if nh % 1000 == 0 else 8
  npass = W1.shape[1] // _F

  epw = e // _W
  b = _batch_size(epw)
  nb = epw // b
  if nb * b == epw:
    src2 = src.reshape(_W, nb, b)
    dst2 = dst.reshape(_W, nb, b)
    src_t = dst_t = None
  else:
    src_w = src.reshape(_W, epw)
    dst_w = dst.reshape(_W, epw)
    src2 = src_w[:, :nb * b].reshape(_W, nb, b)
    dst2 = dst_w[:, :nb * b].reshape(_W, nb, b)
    src_t = src_w[:, nb * b:]
    dst_t = dst_w[:, nb * b:]

  x_pk = x.reshape(nh, 2 * nf)
  w1bd = jnp.stack([_block_diag2(W1[:, p * _F:(p + 1) * _F])
                    for p in range(npass)])
  w2bd = jnp.stack([_block_diag2(W2[p * _F:(p + 1) * _F, :])
                    for p in range(npass)])
  b1pk = jnp.stack([jnp.concatenate([b1[p * _F:(p + 1) * _F]] * 2)
                    for p in range(npass)])[:, None, :]
  b2pk = jnp.concatenate([b2, b2]).reshape(1, 2 * _F)

  deg_parts = _sc_degree(dst2, dst_t, n)
  deg_pk = deg_parts.reshape(NC, nh, 32)
  dinv_pk, h1p_pk = _tc_first(x_pk, w1bd, deg_pk, blkh)
  agg1 = _sc_edge_agg(src2, dst2, src_t, dst_t,
                      h1p_pk.reshape(npass, n, _F))
  h2p_pk = _tc_mid(agg1.reshape(NC, npass, nh, 2 * _F), h1p_pk,
                   dinv_pk, b1pk, w2bd, blkh)
  agg2 = _sc_edge_agg(src2, dst2, src_t, dst_t, h2p_pk.reshape(1, n, _F))
  fin_pk, logp_pk = _tc_last(agg2.reshape(NC, nh, 2 * _F), h2p_pk,
                             dinv_pk, b2pk, blkh)
  return (fin_pk.reshape(n, _F), logp_pk.reshape(n, _F))


# speedup vs baseline: 1.2447x; 1.0008x over previous
import functools

import jax
import jax.numpy as jnp
from jax import lax
from jax.experimental import pallas as pl
from jax.experimental.pallas import tpu as pltpu
from jax.experimental.pallas import tpu_sc as plsc

NC = 2
NS = 16
_W = NC * NS
_D = 6
_F = 64


def _batch_size(epw):
  del epw
  return 128


def _mesh():
  return plsc.VectorSubcoreMesh(
      core_axis_name="c", subcore_axis_name="s",
      num_cores=NC, num_subcores=NS)


def _row_split(n_nodes):
  ra = (n_nodes // NS) // 8 * 8
  return ra, n_nodes - NS * ra


def _sc_degree(dst2, dst_t, n_nodes):
  _, nb, b = dst2.shape
  tail = dst_t.shape[1] if dst_t is not None else 0
  ra, res = _row_split(n_nodes)

  scr = [pltpu.VMEM((nb, b), jnp.int32)]
  if tail:
    scr.append(pltpu.VMEM((tail,), jnp.int32))
  scr += [
      pltpu.VMEM((b, 16), jnp.float32),
      pltpu.VMEM_SHARED((n_nodes, 16), jnp.float32),
      pltpu.SemaphoreType.DMA,
  ]

  @functools.partial(
      pl.kernel,
      out_type=jax.ShapeDtypeStruct((NC, n_nodes, 16), jnp.float32),
      mesh=_mesh(),
      compiler_params=pltpu.CompilerParams(use_tc_tiling_on_sc=False),
      scratch_types=scr,
  )
  def deg_kernel(*refs):
    it = iter(refs)
    dst_hbm = next(it)
    dstt_hbm = next(it) if tail else None
    out_hbm = next(it)
    didx = next(it)
    didx_t = next(it) if tail else None
    ones_v = next(it)
    cnt_sh = next(it)
    sem = next(it)
    cid = lax.axis_index("c")
    sid = lax.axis_index("s")
    wid = sid * NC + cid
    tb = sid * ra

    def fill_row(i, carry):
      ones_v[i, :] = jnp.full((16,), 1.0, jnp.float32)
      return carry
    lax.fori_loop(0, b, fill_row, 0)
    pltpu.sync_copy(dst_hbm.at[wid], didx)
    if tail:
      pltpu.sync_copy(dstt_hbm.at[wid], didx_t)

    done = 0
    while done < ra:
      sz = min(b, ra - done)
      pltpu.sync_copy(ones_v.at[pl.ds(0, sz)],
                      cnt_sh.at[pl.ds(tb + done, sz)])
      done += sz
    if res:
      @pl.when(sid == NS - 1)
      def _():
        pltpu.sync_copy(ones_v.at[pl.ds(0, res)],
                        cnt_sh.at[pl.ds(NS * ra, res)])
    plsc.subcore_barrier()

    def batch(i, carry):
      pltpu.async_copy(ones_v, cnt_sh.at[didx.at[i]], sem, add=True)
      @pl.when(i >= _D)
      def _():
        pltpu.make_async_copy(ones_v, cnt_sh.at[didx.at[i]], sem).wait()
      return carry
    lax.fori_loop(0, nb, batch, 0)
    for d in range(min(_D, nb)):
      pltpu.make_async_copy(ones_v, cnt_sh.at[didx.at[d]], sem).wait()
    if tail:
      pltpu.sync_copy(ones_v.at[pl.ds(0, tail)], cnt_sh.at[didx_t], add=True)
    plsc.subcore_barrier()

    pltpu.sync_copy(cnt_sh.at[pl.ds(tb, ra)],
                    out_hbm.at[cid, pl.ds(tb, ra)])
    if res:
      @pl.when(sid == NS - 1)
      def _():
        pltpu.sync_copy(cnt_sh.at[pl.ds(NS * ra, res)],
                        out_hbm.at[cid, pl.ds(NS * ra, res)])

  args = (dst2, dst_t) if tail else (dst2,)
  return deg_kernel(*args)


def _sc_edge_agg(src2, dst2, src_t, dst_t, hp3):
  npass, n_nodes, f = hp3.shape
  _, nb, b = src2.shape
  tail = src_t.shape[1] if src_t is not None else 0
  ra, res = _row_split(n_nodes)
  kmain = nb // _D
  rem = nb % _D

  scr = [pltpu.VMEM((nb, b), jnp.int32), pltpu.VMEM((nb, b), jnp.int32)]
  if tail:
    scr += [pltpu.VMEM((tail,), jnp.int32), pltpu.VMEM((tail,), jnp.int32)]
  scr += [
      [pltpu.VMEM((b, f), jnp.float32)] * _D,
      [pltpu.SemaphoreType.DMA] * _D,
      [pltpu.SemaphoreType.DMA] * _D,
      pltpu.VMEM_SHARED((n_nodes, f), jnp.float32),
  ]

  @functools.partial(
      pl.kernel,
      out_type=jax.ShapeDtypeStruct((NC, npass, n_nodes, f), jnp.float32),
      mesh=_mesh(),
      compiler_params=pltpu.CompilerParams(use_tc_tiling_on_sc=False),
      scratch_types=scr,
  )
  def agg_kernel(*refs):
    it = iter(refs)
    src_hbm = next(it)
    dst_hbm = next(it)
    srct_hbm = next(it) if tail else None
    dstt_hbm = next(it) if tail else None
    hp_hbm = next(it)
    out_hbm = next(it)
    sidx = next(it)
    didx = next(it)
    sidx_t = next(it) if tail else None
    didx_t = next(it) if tail else None
    rows = next(it)
    gsem = next(it)
    ssem = next(it)
    agg_sh = next(it)
    cid = lax.axis_index("c")
    sid = lax.axis_index("s")
    wid = sid * NC + cid
    tb = sid * ra

    pltpu.sync_copy(src_hbm.at[wid], sidx)
    pltpu.sync_copy(dst_hbm.at[wid], didx)
    if tail:
      pltpu.sync_copy(srct_hbm.at[wid], sidx_t)
      pltpu.sync_copy(dstt_hbm.at[wid], didx_t)

    for p in range(npass):
      hview = hp_hbm.at[p]

      pltpu.sync_copy(hview.at[pl.ds(tb, ra)], agg_sh.at[pl.ds(tb, ra)])
      if res:
        @pl.when(sid == NS - 1)
        def _():
          pltpu.sync_copy(hview.at[pl.ds(NS * ra, res)],
                          agg_sh.at[pl.ds(NS * ra, res)])
      plsc.subcore_barrier()

      def start_gather(i, d):
        pltpu.async_copy(hview.at[sidx.at[i]], rows[d], gsem[d])

      def wait_gather(i, d):
        pltpu.make_async_copy(hview.at[sidx.at[i]], rows[d], gsem[d]).wait()

      def start_scatter(i, d):
        pltpu.async_copy(rows[d], agg_sh.at[didx.at[i]], ssem[d], add=True)

      def wait_scatter(i, d):
        pltpu.make_async_copy(rows[d], agg_sh.at[didx.at[i]], ssem[d]).wait()

      for d in range(min(_D, nb)):
        start_gather(d, d)

      def kbody(k, carry):
        for d in range(_D):
          i = k * _D + d
          wait_gather(i, d)
          start_scatter(i, d)
          @pl.when(i + _D < nb)
          def _():
            wait_scatter(i, d)
            start_gather(i + _D, d)
        return carry
      lax.fori_loop(0, kmain, kbody, 0)
      for d in range(rem):
        i = kmain * _D + d
        wait_gather(i, d)
        start_scatter(i, d)
      for d in range(min(_D, nb)):
        wait_scatter(0, d)
      if tail:
        pltpu.async_copy(hview.at[sidx_t], rows[0].at[pl.ds(0, tail)],
                         gsem[0]).wait()
        pltpu.sync_copy(rows[0].at[pl.ds(0, tail)], agg_sh.at[didx_t],
                        add=True)
      plsc.subcore_barrier()

      pltpu.sync_copy(agg_sh.at[pl.ds(tb, ra)],
                      out_hbm.at[cid, p, pl.ds(tb, ra)])
      if res:
        @pl.when(sid == NS - 1)
        def _():
          pltpu.sync_copy(agg_sh.at[pl.ds(NS * ra, res)],
                          out_hbm.at[cid, p, pl.ds(NS * ra, res)])
      if p + 1 < npass:
        plsc.subcore_barrier()

  args = ((src2, dst2, src_t, dst_t, hp3) if tail
          else (src2, dst2, hp3))
  return agg_kernel(*args)


_TC_PARAMS = pltpu.CompilerParams(
    dimension_semantics=("arbitrary",))


def _tc_first(x_pk, w1bd, deg_pk, blkh):
  nh, nf2 = x_pk.shape
  npass = w1bd.shape[0]

  def body(deg_ref, x_ref, w_ref, dinv_ref, h1p_ref):
    de = deg_ref[0, :, 0:1] + deg_ref[1, :, 0:1] - 1.0
    do = deg_ref[0, :, 16:17] + deg_ref[1, :, 16:17] - 1.0
    dinv = jnp.concatenate(
        [jnp.broadcast_to(lax.rsqrt(de), (blkh, _F)),
         jnp.broadcast_to(lax.rsqrt(do), (blkh, _F))], axis=1)
    dinv_ref[...] = dinv
    for p in range(npass):
      h1p_ref[p] = jnp.dot(x_ref[...], w_ref[p],
                           preferred_element_type=jnp.float32) * dinv

  grid = (nh // blkh,)
  return pl.pallas_call(
      body,
      grid=grid,
      in_specs=[
          pl.BlockSpec((NC, blkh, 32), lambda i: (0, i, 0)),
          pl.BlockSpec((blkh, nf2), lambda i: (i, 0)),
          pl.BlockSpec((npass, nf2, 2 * _F), lambda i: (0, 0, 0)),
      ],
      out_specs=[
          pl.BlockSpec((blkh, 2 * _F), lambda i: (i, 0)),
          pl.BlockSpec((npass, blkh, 2 * _F), lambda i: (0, i, 0)),
      ],
      out_shape=[
          jax.ShapeDtypeStruct((nh, 2 * _F), jnp.float32),
          jax.ShapeDtypeStruct((npass, nh, 2 * _F), jnp.float32),
      ],
      compiler_params=_TC_PARAMS,
  )(deg_pk, x_pk, w1bd)


def _tc_mid(agg1_pk, h1p_pk, dinv_pk, b1pk, w2bd, blkh):
  npass, nh, _ = h1p_pk.shape

  def body(a_ref, hp_ref, dinv_ref, b_ref, w_ref, out_ref):
    dinv = dinv_ref[...]
    acc = None
    for p in range(npass):
      s = a_ref[0, p] + a_ref[1, p] - hp_ref[p]
      h1 = jnp.maximum(s * dinv + b_ref[p], 0.0)
      t = jnp.dot(h1, w_ref[p], preferred_element_type=jnp.float32)
      acc = t if acc is None else acc + t
    out_ref[...] = acc * dinv

  grid = (nh // blkh,)
  return pl.pallas_call(
      body,
      grid=grid,
      in_specs=[
          pl.BlockSpec((NC, npass, blkh, 2 * _F), lambda i: (0, 0, i, 0)),
          pl.BlockSpec((npass, blkh, 2 * _F), lambda i: (0, i, 0)),
          pl.BlockSpec((blkh, 2 * _F), lambda i: (i, 0)),
          pl.BlockSpec((npass, 1, 2 * _F), lambda i: (0, 0, 0)),
          pl.BlockSpec((npass, 2 * _F, 2 * _F), lambda i: (0, 0, 0)),
      ],
      out_specs=pl.BlockSpec((blkh, 2 * _F), lambda i: (i, 0)),
      out_shape=jax.ShapeDtypeStruct((nh, 2 * _F), jnp.float32),
      compiler_params=_TC_PARAMS,
  )(agg1_pk, h1p_pk, dinv_pk, b1pk, w2bd)


def _tc_last(agg2_pk, h2p_pk, dinv_pk, b2pk, blkh):
  nh, _ = h2p_pk.shape

  def lsm(f):
    m = jnp.max(f, axis=1, keepdims=True)
    sh = f - m
    return sh - jnp.log(jnp.sum(jnp.exp(sh), axis=1, keepdims=True))

  def body(a_ref, hp_ref, dinv_ref, b_ref, fin_ref, logp_ref):
    s = a_ref[0] + a_ref[1] - hp_ref[...]
    fin = s * dinv_ref[...] + b_ref[...]
    fin_ref[...] = fin
    logp_ref[...] = jnp.concatenate(
        [lsm(fin[:, :_F]), lsm(fin[:, _F:])], axis=1)

  grid = (nh // blkh,)
  return pl.pallas_call(
      body,
      grid=grid,
      in_specs=[
          pl.BlockSpec((NC, blkh, 2 * _F), lambda i: (0, i, 0)),
          pl.BlockSpec((blkh, 2 * _F), lambda i: (i, 0)),
          pl.BlockSpec((blkh, 2 * _F), lambda i: (i, 0)),
          pl.BlockSpec((1, 2 * _F), lambda i: (0, 0)),
      ],
      out_specs=[
          pl.BlockSpec((blkh, 2 * _F), lambda i: (i, 0)),
          pl.BlockSpec((blkh, 2 * _F), lambda i: (i, 0)),
      ],
      out_shape=[
          jax.ShapeDtypeStruct((nh, 2 * _F), jnp.float32),
          jax.ShapeDtypeStruct((nh, 2 * _F), jnp.float32),
      ],
      compiler_params=_TC_PARAMS,
  )(agg2_pk, h2p_pk, dinv_pk, b2pk)


def _block_diag2(w):
  k = w.shape[0]
  z = jnp.zeros((k, _F), jnp.float32)
  return jnp.concatenate(
      [jnp.concatenate([w, z], axis=1),
       jnp.concatenate([z, w], axis=1)], axis=0)


def kernel(x, edge_index, W1, b1, W2, b2):
  n = x.shape[0]
  e = edge_index.shape[1]
  nf = x.shape[1]
  src = edge_index[0]
  dst = edge_index[1]
  nh = n // 2
  blkh = 1000 if nh % 1000 == 0 else 8
  npass = W1.shape[1] // _F

  epw = e // _W
  b = _batch_size(epw)
  nb = epw // b
  if nb * b == epw:
    src2 = src.reshape(_W, nb, b)
    dst2 = dst.reshape(_W, nb, b)
    src_t = dst_t = None
  else:
    src_w = src.reshape(_W, epw)
    dst_w = dst.reshape(_W, epw)
    src2 = src_w[:, :nb * b].reshape(_W, nb, b)
    dst2 = dst_w[:, :nb * b].reshape(_W, nb, b)
    src_t = src_w[:, nb * b:]
    dst_t = dst_w[:, nb * b:]

  x_pk = x.reshape(nh, 2 * nf)
  w1bd = jnp.stack([_block_diag2(W1[:, p * _F:(p + 1) * _F])
                    for p in range(npass)])
  w2bd = jnp.stack([_block_diag2(W2[p * _F:(p + 1) * _F, :])
                    for p in range(npass)])
  b1pk = jnp.stack([jnp.concatenate([b1[p * _F:(p + 1) * _F]] * 2)
                    for p in range(npass)])[:, None, :]
  b2pk = jnp.concatenate([b2, b2]).reshape(1, 2 * _F)

  deg_parts = _sc_degree(dst2, dst_t, n)
  deg_pk = deg_parts.reshape(NC, nh, 32)
  dinv_pk, h1p_pk = _tc_first(x_pk, w1bd, deg_pk, blkh)
  agg1 = _sc_edge_agg(src2, dst2, src_t, dst_t,
                      h1p_pk.reshape(npass, n, _F))
  h2p_pk = _tc_mid(agg1.reshape(NC, npass, nh, 2 * _F), h1p_pk,
                   dinv_pk, b1pk, w2bd, blkh)
  agg2 = _sc_edge_agg(src2, dst2, src_t, dst_t, h2p_pk.reshape(1, n, _F))
  fin_pk, logp_pk = _tc_last(agg2.reshape(NC, nh, 2 * _F), h2p_pk,
                             dinv_pk, b2pk, blkh)
  return (fin_pk.reshape(n, _F), logp_pk.reshape(n, _F))
